# Initial kernel scaffold; baseline (speedup 1.0000x reference)
import jax
import jax.numpy as jnp
from jax.experimental import pallas as pl


def kernel(embeddings):
    raise NotImplementedError("write your pallas kernel here")


# speedup vs baseline: 2.3132x; 2.3132x over previous
import jax
import jax.numpy as jnp
from jax.experimental import pallas as pl
from jax.experimental.pallas import tpu as pltpu

N = 4096
D = 128
BLK = 512
NBLK = N // BLK


def _koleo_kernel(emb_blk_ref, emb_ref, out_ref, rowmin_ref, gmax_ref):
    i = pl.program_id(0)
    emb = emb_ref[...]
    blk = emb_blk_ref[...]

    sqn_all = jnp.sum(emb * emb, axis=1)[None, :]
    sqn_blk = jnp.sum(blk * blk, axis=1)[:, None]

    dot = jax.lax.dot_general(
        blk, emb, (((1,), (1,)), ((), ())),
        preferred_element_type=jnp.float32)

    sq = sqn_blk + sqn_all - 2.0 * dot
    masked = jnp.where(sq <= 0.0, jnp.inf, sq)
    rowmin = jnp.min(masked, axis=1)[:, None]
    tilemax = jnp.max(sq)

    rowmin_ref[pl.ds(i, 1), :] = rowmin.reshape(1, BLK)

    @pl.when(i == 0)
    def _():
        gmax_ref[0, 0] = tilemax

    @pl.when(i > 0)
    def _():
        gmax_ref[0, 0] = jnp.maximum(gmax_ref[0, 0], tilemax)

    @pl.when(i == NBLK - 1)
    def _():
        g = jnp.maximum(gmax_ref[0, 0], 0.0)
        m = jnp.minimum(rowmin_ref[...], g)
        d = jnp.sqrt(m)
        out_ref[...] = jnp.reshape((-1.0 / N) * jnp.sum(jnp.log(d)), (1, 1))


def kernel(embeddings):
    out = pl.pallas_call(
        _koleo_kernel,
        grid=(NBLK,),
        in_specs=[
            pl.BlockSpec((BLK, D), lambda i: (i, 0)),
            pl.BlockSpec((N, D), lambda i: (0, 0)),
        ],
        out_specs=pl.BlockSpec((1, 1), lambda i: (0, 0)),
        out_shape=jax.ShapeDtypeStruct((1, 1), jnp.float32),
        scratch_shapes=[
            pltpu.VMEM((NBLK, BLK), jnp.float32),
            pltpu.SMEM((1, 1), jnp.float32),
        ],
    )(embeddings, embeddings)
    return out[0, 0]
